# Initial kernel scaffold; baseline (speedup 1.0000x reference)
import jax
import jax.numpy as jnp
from jax.experimental import pallas as pl


def kernel(vertex, edges, degE, degV, user_emb, item_emb, W1, b1, W2, b2):
    raise NotImplementedError("write your pallas kernel here")


# speedup vs baseline: 1.1704x; 1.1704x over previous
import functools

import jax
import jax.numpy as jnp
from jax import lax
from jax.experimental import pallas as pl
from jax.experimental.pallas import tpu as pltpu
from jax.experimental.pallas import tpu_sc as plsc

_NU = 5000
_NI = 5000
_N = 10000
_E = 160000
_NNZ = 320000
_D = 256

_NC = 2
_NS = 16

_S = 32
_W = 8
_NPAD = 10016
_EPAD = 163840
_AVROWS = 10240
_NNZP = 327680
_PPT = _NNZP // _NS
_CH = 1024
_NCHUNK = _PPT // _CH
_IDXR = _CH // 128
_ECHT = _E // _NS
_ECH = 2000
_NECH = _ECHT // _ECH
_EZT = _EPAD // _NS
_SCH = 1024
_ZR = 320
_AVZT = _AVROWS // _NS
_OV0 = 624
_OV1 = 640


def _sc_body(compute_w, *refs):
    if compute_w:
        (xsl, vidx, eidx, dege, out, w_out,
         acc_e, acc_v, w_sh, rows, zrows, zbuf, ones,
         vi2, ei2, va2, wcv, dcv, rws, sem1, sem2) = refs
    else:
        (xsl, vidx, eidx, w_in, out,
         acc_e, acc_v, w_sh, rows, zrows, zbuf, ones,
         vi2, ei2, va2, wcv, dcv, rws, sem1, sem2) = refs

    c = lax.axis_index("c")
    t = lax.axis_index("s")
    iota = lax.iota(jnp.int32, 16)

    zf = jnp.zeros((16,), jnp.float32)
    of = jnp.ones((16,), jnp.float32)

    def _fill(i, _):
        zbuf[pl.ds(i * 16, 16)] = zf
        return 0
    lax.fori_loop(0, 40, _fill, 0)
    for i in range(8):
        ones[pl.ds(i * 16, 16)] = of
    for i in range(_ZR // 16):
        pltpu.sync_copy(xsl.at[pl.ds(_N, 16), :],
                        zrows.at[pl.ds(i * 16, 16), :])

    if compute_w:
        def _zw(m, _):
            pltpu.sync_copy(zbuf, w_sh.at[pl.ds(t * _EZT + m * 640, 640)])
            return 0
        lax.fori_loop(0, _EZT // 640, _zw, 0)
        plsc.subcore_barrier()

        def _cnt(k, _):
            r0 = t * (_PPT // 128) + k * _IDXR
            pltpu.sync_copy(eidx.at[pl.ds(r0, _IDXR), :], ei2)
            hs = [pltpu.async_copy(ones.at[pl.ds(0, 128)],
                                   w_sh.at[ei2.at[j]], sem2, add=True)
                  for j in range(_IDXR)]
            for h in hs:
                h.wait()
            return 0
        lax.fori_loop(0, _NCHUNK, _cnt, 0)
        plsc.subcore_barrier()

        def _wc(k, _):
            b = t * _ECHT + k * _ECH
            pltpu.sync_copy(w_sh.at[pl.ds(b, _ECH)], wcv.at[pl.ds(0, _ECH)])
            pltpu.sync_copy(dege.at[pl.ds(b, _ECH)], dcv)

            def _wv(j, _):
                cv = wcv[pl.ds(j * 16, 16)]
                dv = dcv[pl.ds(j * 16, 16)]
                wcv[pl.ds(j * 16, 16)] = dv / jnp.maximum(cv, 1.0)
                return 0
            lax.fori_loop(0, _ECH // 16, _wv, 0)
            pltpu.sync_copy(wcv.at[pl.ds(0, _ECH)], w_sh.at[pl.ds(b, _ECH)])
            pltpu.sync_copy(wcv.at[pl.ds(0, _ECH)], w_out.at[pl.ds(b, _ECH)])
            return 0
        lax.fori_loop(0, _NECH, _wc, 0)
        pltpu.sync_copy(zbuf.at[pl.ds(0, 240)],
                        w_sh.at[pl.ds(_E + t * 240, 240)])
        plsc.subcore_barrier()
    else:
        def _wl(k, _):
            b = t * _ECHT + k * _ECH
            pltpu.sync_copy(w_in.at[pl.ds(b, _ECH)], wcv.at[pl.ds(0, _ECH)])
            pltpu.sync_copy(wcv.at[pl.ds(0, _ECH)], w_sh.at[pl.ds(b, _ECH)])
            return 0
        lax.fori_loop(0, _NECH, _wl, 0)
        pltpu.sync_copy(zbuf.at[pl.ds(0, 240)],
                        w_sh.at[pl.ds(_E + t * 240, 240)])
        plsc.subcore_barrier()

    def _slice(s_local, _):
        s = c * 16 + s_local

        def _za(m, _):
            pltpu.sync_copy(zrows, acc_e.at[pl.ds(t * _EZT + m * _ZR, _ZR), :])
            return 0
        lax.fori_loop(0, _EZT // _ZR, _za, 0)
        pltpu.sync_copy(zrows, acc_v.at[pl.ds(t * _AVZT, _ZR), :])
        pltpu.sync_copy(zrows, acc_v.at[pl.ds(t * _AVZT + _ZR, _ZR), :])
        plsc.subcore_barrier()

        off = s * _NPAD

        def _pha(k, _):
            r0 = t * (_PPT // 128) + k * _IDXR
            pltpu.sync_copy(vidx.at[pl.ds(r0, _IDXR), :], vi2)

            def _adj(r, _):
                for l in range(8):
                    va2[r, pl.ds(l * 16, 16)] = vi2[r, pl.ds(l * 16, 16)] + off
                return 0
            lax.fori_loop(0, _IDXR, _adj, 0)

            hs = [pltpu.async_copy(xsl.at[va2.at[j]],
                                   rows.at[pl.ds(j * 128, 128), :], sem1)
                  for j in range(_IDXR)]
            pltpu.sync_copy(eidx.at[pl.ds(r0, _IDXR), :], ei2)
            for h in hs:
                h.wait()
            hs2 = [pltpu.async_copy(rows.at[pl.ds(j * 128, 128), :],
                                    acc_e.at[ei2.at[j]], sem2, add=True)
                   for j in range(_IDXR)]
            for h in hs2:
                h.wait()
            return 0
        lax.fori_loop(0, _NCHUNK, _pha, 0)
        plsc.subcore_barrier()

        def _scl(k, _):
            b = t * _EZT + k * _SCH
            pltpu.sync_copy(w_sh.at[pl.ds(b, _SCH)], wcv.at[pl.ds(0, _SCH)])
            pltpu.sync_copy(acc_e.at[pl.ds(b, 512), :],
                            rws.at[:, pl.ds(0, 8)])
            pltpu.sync_copy(acc_e.at[pl.ds(b + 512, 512), :],
                            rws.at[:, pl.ds(8, 8)])

            def _sv(g, _):
                wa = wcv[pl.ds(g * 16, 16)]
                wb = wcv[pl.ds(512 + g * 16, 16)]
                for kk in range(16):
                    r = g * 16 + kk
                    wv = jnp.where(iota < 8, wa[kk], wb[kk])
                    rws[r, :] = rws[r, :] * wv
                return 0
            lax.fori_loop(0, 32, _sv, 0)
            pltpu.sync_copy(rws.at[:, pl.ds(0, 8)],
                            acc_e.at[pl.ds(b, 512), :])
            pltpu.sync_copy(rws.at[:, pl.ds(8, 8)],
                            acc_e.at[pl.ds(b + 512, 512), :])
            return 0
        lax.fori_loop(0, _EZT // _SCH, _scl, 0)
        plsc.subcore_barrier()

        def _phb(k, _):
            r0 = t * (_PPT // 128) + k * _IDXR
            pltpu.sync_copy(eidx.at[pl.ds(r0, _IDXR), :], ei2)
            hs = [pltpu.async_copy(acc_e.at[ei2.at[j]],
                                   rows.at[pl.ds(j * 128, 128), :], sem1)
                  for j in range(_IDXR)]
            pltpu.sync_copy(vidx.at[pl.ds(r0, _IDXR), :], vi2)
            for h in hs:
                h.wait()
            hs2 = [pltpu.async_copy(rows.at[pl.ds(j * 128, 128), :],
                                    acc_v.at[vi2.at[j]], sem2, add=True)
                   for j in range(_IDXR)]
            for h in hs2:
                h.wait()
            return 0
        lax.fori_loop(0, _NCHUNK, _phb, 0)
        plsc.subcore_barrier()

        pltpu.sync_copy(acc_v.at[pl.ds(t * _OV0, _OV0), :],
                        out.at[s, pl.ds(t * _OV0, _OV0), :])

        @pl.when(t == _NS - 1)
        def _tail():
            pltpu.sync_copy(acc_v.at[pl.ds(_NS * _OV0, _OV1 - _OV0), :],
                            out.at[s, pl.ds(_NS * _OV0, _OV1 - _OV0), :])
        plsc.subcore_barrier()
        return 0

    lax.fori_loop(0, 16, _slice, 0)


def _make_sc_kernel(compute_w):
    mesh = plsc.VectorSubcoreMesh(core_axis_name="c", subcore_axis_name="s",
                                  num_cores=_NC, num_subcores=_NS)
    out_type = [jax.ShapeDtypeStruct((_S, _N, _W), jnp.float32)]
    if compute_w:
        out_type.append(jax.ShapeDtypeStruct((_E,), jnp.float32))
    scratch = [
        pltpu.VMEM_SHARED((_EPAD, _W), jnp.float32),
        pltpu.VMEM_SHARED((_AVROWS, _W), jnp.float32),
        pltpu.VMEM_SHARED((_EPAD,), jnp.float32),
        pltpu.VMEM((_CH, _W), jnp.float32),
        pltpu.VMEM((_ZR, _W), jnp.float32),
        pltpu.VMEM((640,), jnp.float32),
        pltpu.VMEM((128,), jnp.float32),
        pltpu.VMEM((_IDXR, 128), jnp.int32),
        pltpu.VMEM((_IDXR, 128), jnp.int32),
        pltpu.VMEM((_IDXR, 128), jnp.int32),
        pltpu.VMEM((_ECH,), jnp.float32),
        pltpu.VMEM((_ECH,), jnp.float32),
        pltpu.VMEM((512, 16), jnp.float32),
        pltpu.SemaphoreType.DMA,
        pltpu.SemaphoreType.DMA,
    ]
    return pl.kernel(functools.partial(_sc_body, compute_w),
                     out_type=tuple(out_type), mesh=mesh,
                     scratch_types=scratch,
                     compiler_params=pltpu.CompilerParams(
                         use_tc_tiling_on_sc=False))


_sc_layer1 = _make_sc_kernel(True)
_sc_layer2 = _make_sc_kernel(False)


def _tc_body(x_ref, xv_ref, dv_ref, w_ref, b_ref, o_ref):
    a = x_ref[...] + xv_ref[...] * dv_ref[...]
    y = jnp.dot(a, w_ref[...], preferred_element_type=jnp.float32)
    y = y + b_ref[...]
    nrm = jnp.sqrt(jnp.sum(y * y, axis=1, keepdims=True)) + 1e-12
    o_ref[...] = jnp.maximum(y / nrm, 0.0)


_TC_BS = 1000


def _tc_layer(x, xv, degv2, w, b):
    grid = (_N // _TC_BS,)
    return pl.pallas_call(
        _tc_body,
        grid=grid,
        in_specs=[
            pl.BlockSpec((_TC_BS, _D), lambda i: (i, 0)),
            pl.BlockSpec((_TC_BS, _D), lambda i: (i, 0)),
            pl.BlockSpec((_TC_BS, 1), lambda i: (i, 0)),
            pl.BlockSpec((_D, _D), lambda i: (0, 0)),
            pl.BlockSpec((1, _D), lambda i: (0, 0)),
        ],
        out_specs=pl.BlockSpec((_TC_BS, _D), lambda i: (i, 0)),
        out_shape=jax.ShapeDtypeStruct((_N, _D), jnp.float32),
    )(x, xv, degv2, w, b)


def _slice_layout(x):
    xs = x.reshape(_N, _S, _W).transpose(1, 0, 2)
    xs = jnp.pad(xs, ((0, 0), (0, _NPAD - _N), (0, 0)))
    return xs.reshape(_S * _NPAD, _W)


def _unslice(o):
    return o.transpose(1, 0, 2).reshape(_N, _D)


def kernel(vertex, edges, degE, degV, user_emb, item_emb, W1, b1, W2, b2):
    x = jnp.concatenate([user_emb, item_emb], axis=0)

    npad = _NNZP - _NNZ
    pad_i = jnp.arange(npad, dtype=jnp.int32)
    vp = jnp.concatenate([vertex, _N + (pad_i % 8)]).reshape(_NNZP // 128, 128)
    ep = jnp.concatenate([edges, _E + (pad_i % 2048)]).reshape(_NNZP // 128, 128)
    degv2 = degV.reshape(_N, 1)
    b1r = b1.reshape(1, _D)
    b2r = b2.reshape(1, _D)

    xv1_sl, w = _sc_layer1(_slice_layout(x), vp, ep, degE)
    x1 = _tc_layer(x, _unslice(xv1_sl), degv2, W1, b1r)
    (xv2_sl,) = _sc_layer2(_slice_layout(x1), vp, ep, w)
    x2 = _tc_layer(x1, _unslice(xv2_sl), degv2, W2, b2r)
    return (x2[:_NU], x2[_NU:])


# speedup vs baseline: 1.3995x; 1.1958x over previous
import functools

import jax
import jax.numpy as jnp
from jax import lax
from jax.experimental import pallas as pl
from jax.experimental.pallas import tpu as pltpu
from jax.experimental.pallas import tpu_sc as plsc

_NU = 5000
_NI = 5000
_N = 10000
_E = 160000
_NNZ = 320000
_D = 256

_NC = 2
_NS = 16

_S = 32
_W = 8
_XROWS = (_N + 8) * _S
_EPAD = 163840
_AVROWS = 10240
_NNZP = 327680
_PPT = _NNZP // _NS
_CH = 1024
_NCHUNK = _PPT // _CH
_IDXR = _CH // 128
_ECHT = _E // _NS
_ECH = 2000
_NECH = _ECHT // _ECH
_EZT = _EPAD // _NS
_SCH = 1024
_ZR = 320
_AVZT = _AVROWS // _NS
_OV0 = 624
_OV1 = 640


def _sc_body(compute_w, *refs):
    if compute_w:
        (xsl, vidx, vidx32, eidx, dege, out, w_out,
         acc_e, acc_v, w_sh, rows, zrows, zbuf, ones,
         vi2, ei2, va2, wcv, dcv, rws, sem1, sem2) = refs
    else:
        (xsl, vidx, vidx32, eidx, w_in, out,
         acc_e, acc_v, w_sh, rows, zrows, zbuf, ones,
         vi2, ei2, va2, wcv, dcv, rws, sem1, sem2) = refs

    c = lax.axis_index("c")
    t = lax.axis_index("s")
    iota = lax.iota(jnp.int32, 16)

    zf = jnp.zeros((16,), jnp.float32)
    of = jnp.ones((16,), jnp.float32)

    def _fill(i, _):
        zbuf[pl.ds(i * 16, 16)] = zf
        return 0
    lax.fori_loop(0, 40, _fill, 0)
    for i in range(8):
        ones[pl.ds(i * 16, 16)] = of
    for i in range(_ZR // 16):
        pltpu.sync_copy(xsl.at[pl.ds(_N * _S, 16), :],
                        zrows.at[pl.ds(i * 16, 16), :])

    if compute_w:
        def _zw(m, _):
            pltpu.sync_copy(zbuf, w_sh.at[pl.ds(t * _EZT + m * 640, 640)])
            return 0
        lax.fori_loop(0, _EZT // 640, _zw, 0)
        plsc.subcore_barrier()

        def _cnt(k, _):
            r0 = t * (_PPT // 128) + k * _IDXR
            pltpu.sync_copy(eidx.at[pl.ds(r0, _IDXR), :], ei2)
            hs = [pltpu.async_copy(ones.at[pl.ds(0, 128)],
                                   w_sh.at[ei2.at[j]], sem2, add=True)
                  for j in range(_IDXR)]
            for h in hs:
                h.wait()
            return 0
        lax.fori_loop(0, _NCHUNK, _cnt, 0)
        plsc.subcore_barrier()

        def _wc(k, _):
            b = t * _ECHT + k * _ECH
            pltpu.sync_copy(w_sh.at[pl.ds(b, _ECH)], wcv.at[pl.ds(0, _ECH)])
            pltpu.sync_copy(dege.at[pl.ds(b, _ECH)], dcv)

            def _wv(j, _):
                cv = wcv[pl.ds(j * 16, 16)]
                dv = dcv[pl.ds(j * 16, 16)]
                wcv[pl.ds(j * 16, 16)] = dv / jnp.maximum(cv, 1.0)
                return 0
            lax.fori_loop(0, _ECH // 16, _wv, 0)
            pltpu.sync_copy(wcv.at[pl.ds(0, _ECH)], w_sh.at[pl.ds(b, _ECH)])
            pltpu.sync_copy(wcv.at[pl.ds(0, _ECH)], w_out.at[pl.ds(b, _ECH)])
            return 0
        lax.fori_loop(0, _NECH, _wc, 0)
        pltpu.sync_copy(zbuf.at[pl.ds(0, 240)],
                        w_sh.at[pl.ds(_E + t * 240, 240)])
        plsc.subcore_barrier()
    else:
        def _wl(k, _):
            b = t * _ECHT + k * _ECH
            pltpu.sync_copy(w_in.at[pl.ds(b, _ECH)], wcv.at[pl.ds(0, _ECH)])
            pltpu.sync_copy(wcv.at[pl.ds(0, _ECH)], w_sh.at[pl.ds(b, _ECH)])
            return 0
        lax.fori_loop(0, _NECH, _wl, 0)
        pltpu.sync_copy(zbuf.at[pl.ds(0, 240)],
                        w_sh.at[pl.ds(_E + t * 240, 240)])
        plsc.subcore_barrier()

    def _slice(s_local, _):
        s = c * 16 + s_local

        def _za(m, _):
            pltpu.sync_copy(zrows, acc_e.at[pl.ds(t * _EZT + m * _ZR, _ZR), :])
            return 0
        lax.fori_loop(0, _EZT // _ZR, _za, 0)
        pltpu.sync_copy(zrows, acc_v.at[pl.ds(t * _AVZT, _ZR), :])
        pltpu.sync_copy(zrows, acc_v.at[pl.ds(t * _AVZT + _ZR, _ZR), :])
        plsc.subcore_barrier()

        off = s

        def _pha(k, _):
            r0 = t * (_PPT // 128) + k * _IDXR
            pltpu.sync_copy(vidx32.at[pl.ds(r0, _IDXR), :], vi2)

            def _adj(r, _):
                for l in range(8):
                    va2[r, pl.ds(l * 16, 16)] = vi2[r, pl.ds(l * 16, 16)] + off
                return 0
            lax.fori_loop(0, _IDXR, _adj, 0)

            hs = [pltpu.async_copy(xsl.at[va2.at[j]],
                                   rows.at[pl.ds(j * 128, 128), :], sem1)
                  for j in range(_IDXR)]
            pltpu.sync_copy(eidx.at[pl.ds(r0, _IDXR), :], ei2)
            for h in hs:
                h.wait()
            hs2 = [pltpu.async_copy(rows.at[pl.ds(j * 128, 128), :],
                                    acc_e.at[ei2.at[j]], sem2, add=True)
                   for j in range(_IDXR)]
            for h in hs2:
                h.wait()
            return 0
        lax.fori_loop(0, _NCHUNK, _pha, 0)
        plsc.subcore_barrier()

        def _scl(k, _):
            b = t * _EZT + k * _SCH
            pltpu.sync_copy(w_sh.at[pl.ds(b, _SCH)], wcv.at[pl.ds(0, _SCH)])
            pltpu.sync_copy(acc_e.at[pl.ds(b, 512), :],
                            rws.at[:, pl.ds(0, 8)])
            pltpu.sync_copy(acc_e.at[pl.ds(b + 512, 512), :],
                            rws.at[:, pl.ds(8, 8)])

            def _sv(g, _):
                wa = wcv[pl.ds(g * 16, 16)]
                wb = wcv[pl.ds(512 + g * 16, 16)]
                for kk in range(16):
                    r = g * 16 + kk
                    wv = jnp.where(iota < 8, wa[kk], wb[kk])
                    rws[r, :] = rws[r, :] * wv
                return 0
            lax.fori_loop(0, 32, _sv, 0)
            pltpu.sync_copy(rws.at[:, pl.ds(0, 8)],
                            acc_e.at[pl.ds(b, 512), :])
            pltpu.sync_copy(rws.at[:, pl.ds(8, 8)],
                            acc_e.at[pl.ds(b + 512, 512), :])
            return 0
        lax.fori_loop(0, _EZT // _SCH, _scl, 0)
        plsc.subcore_barrier()

        def _phb(k, _):
            r0 = t * (_PPT // 128) + k * _IDXR
            pltpu.sync_copy(eidx.at[pl.ds(r0, _IDXR), :], ei2)
            hs = [pltpu.async_copy(acc_e.at[ei2.at[j]],
                                   rows.at[pl.ds(j * 128, 128), :], sem1)
                  for j in range(_IDXR)]
            pltpu.sync_copy(vidx.at[pl.ds(r0, _IDXR), :], vi2)
            for h in hs:
                h.wait()
            hs2 = [pltpu.async_copy(rows.at[pl.ds(j * 128, 128), :],
                                    acc_v.at[vi2.at[j]], sem2, add=True)
                   for j in range(_IDXR)]
            for h in hs2:
                h.wait()
            return 0
        lax.fori_loop(0, _NCHUNK, _phb, 0)
        plsc.subcore_barrier()

        pltpu.sync_copy(acc_v.at[pl.ds(t * _OV0, _OV0), :],
                        out.at[pl.ds(t * _OV0, _OV0), s, :])

        @pl.when(t == _NS - 1)
        def _tail():
            pltpu.sync_copy(acc_v.at[pl.ds(_NS * _OV0, _OV1 - _OV0), :],
                            out.at[pl.ds(_NS * _OV0, _OV1 - _OV0), s, :])
        plsc.subcore_barrier()
        return 0

    lax.fori_loop(0, 16, _slice, 0)


def _make_sc_kernel(compute_w):
    mesh = plsc.VectorSubcoreMesh(core_axis_name="c", subcore_axis_name="s",
                                  num_cores=_NC, num_subcores=_NS)
    out_type = [jax.ShapeDtypeStruct((_N, _S, _W), jnp.float32)]
    if compute_w:
        out_type.append(jax.ShapeDtypeStruct((_E,), jnp.float32))
    scratch = [
        pltpu.VMEM_SHARED((_EPAD, _W), jnp.float32),
        pltpu.VMEM_SHARED((_AVROWS, _W), jnp.float32),
        pltpu.VMEM_SHARED((_EPAD,), jnp.float32),
        pltpu.VMEM((_CH, _W), jnp.float32),
        pltpu.VMEM((_ZR, _W), jnp.float32),
        pltpu.VMEM((640,), jnp.float32),
        pltpu.VMEM((128,), jnp.float32),
        pltpu.VMEM((_IDXR, 128), jnp.int32),
        pltpu.VMEM((_IDXR, 128), jnp.int32),
        pltpu.VMEM((_IDXR, 128), jnp.int32),
        pltpu.VMEM((_ECH,), jnp.float32),
        pltpu.VMEM((_ECH,), jnp.float32),
        pltpu.VMEM((512, 16), jnp.float32),
        pltpu.SemaphoreType.DMA,
        pltpu.SemaphoreType.DMA,
    ]
    return pl.kernel(functools.partial(_sc_body, compute_w),
                     out_type=tuple(out_type), mesh=mesh,
                     scratch_types=scratch,
                     compiler_params=pltpu.CompilerParams(
                         use_tc_tiling_on_sc=False))


_sc_layer1 = _make_sc_kernel(True)
_sc_layer2 = _make_sc_kernel(False)


def _tc_body(x_ref, xv_ref, dv_ref, w_ref, b_ref, o_ref):
    a = x_ref[...] + xv_ref[...] * dv_ref[...]
    y = jnp.dot(a, w_ref[...], preferred_element_type=jnp.float32)
    y = y + b_ref[...]
    nrm = jnp.sqrt(jnp.sum(y * y, axis=1, keepdims=True)) + 1e-12
    o_ref[...] = jnp.maximum(y / nrm, 0.0)


_TC_BS = 1000


def _tc_layer(x, xv, degv2, w, b):
    grid = (_N // _TC_BS,)
    return pl.pallas_call(
        _tc_body,
        grid=grid,
        in_specs=[
            pl.BlockSpec((_TC_BS, _D), lambda i: (i, 0)),
            pl.BlockSpec((_TC_BS, _D), lambda i: (i, 0)),
            pl.BlockSpec((_TC_BS, 1), lambda i: (i, 0)),
            pl.BlockSpec((_D, _D), lambda i: (0, 0)),
            pl.BlockSpec((1, _D), lambda i: (0, 0)),
        ],
        out_specs=pl.BlockSpec((_TC_BS, _D), lambda i: (i, 0)),
        out_shape=jax.ShapeDtypeStruct((_N, _D), jnp.float32),
    )(x, xv, degv2, w, b)


def _flat_layout(x):
    return jnp.pad(x, ((0, 8), (0, 0))).reshape(_XROWS, _W)


def kernel(vertex, edges, degE, degV, user_emb, item_emb, W1, b1, W2, b2):
    x = jnp.concatenate([user_emb, item_emb], axis=0)

    npad = _NNZP - _NNZ
    pad_i = jnp.arange(npad, dtype=jnp.int32)
    vpad = _N + (pad_i % 8)
    vfull = jnp.concatenate([vertex, vpad])
    vp = vfull.reshape(_NNZP // 128, 128)
    vp32 = (vfull * _S).reshape(_NNZP // 128, 128)
    ep = jnp.concatenate([edges, _E + (pad_i % 2048)]).reshape(_NNZP // 128, 128)
    degv2 = degV.reshape(_N, 1)
    b1r = b1.reshape(1, _D)
    b2r = b2.reshape(1, _D)

    xv1, w = _sc_layer1(_flat_layout(x), vp, vp32, ep, degE)
    x1 = _tc_layer(x, xv1.reshape(_N, _D), degv2, W1, b1r)
    (xv2,) = _sc_layer2(_flat_layout(x1), vp, vp32, ep, w)
    x2 = _tc_layer(x1, xv2.reshape(_N, _D), degv2, W2, b2r)
    return (x2[:_NU], x2[_NU:])


# speedup vs baseline: 1.4355x; 1.0257x over previous
import functools

import jax
import jax.numpy as jnp
from jax import lax
from jax.experimental import pallas as pl
from jax.experimental.pallas import tpu as pltpu
from jax.experimental.pallas import tpu_sc as plsc

_NU = 5000
_NI = 5000
_N = 10000
_E = 160000
_NNZ = 320000
_D = 256

_NC = 2
_NS = 16

_S = 32
_W = 8
_XROWS = (_N + 8) * _S
_EPAD = 163840
_AVROWS = 10240
_NNZP = 327680
_PPT = _NNZP // _NS
_CH = 1024
_NCHUNK = _PPT // _CH
_IDXR = _CH // 128
_ECHT = _E // _NS
_ECH = 2000
_NECH = _ECHT // _ECH
_EZT = _EPAD // _NS
_SCH = 1024
_ZR = 320
_AVZT = _AVROWS // _NS
_OV0 = 624
_OV1 = 640


def _sc_body(compute_w, *refs):
    if compute_w:
        (xsl, vidx, vidx32, eidx, dege, out, w_out,
         acc_e, acc_v, w_sh, rows, zrows, zbuf, ones,
         vi2, ei2, va2, wcv, dcv, rws, sem1, sem2) = refs
    else:
        (xsl, vidx, vidx32, eidx, w_in, out,
         acc_e, acc_v, w_sh, rows, zrows, zbuf, ones,
         vi2, ei2, va2, wcv, dcv, rws, sem1, sem2) = refs

    c = lax.axis_index("c")
    t = lax.axis_index("s")
    iota = lax.iota(jnp.int32, 16)

    zf = jnp.zeros((16,), jnp.float32)
    of = jnp.ones((16,), jnp.float32)

    def _fill(i, _):
        zbuf[pl.ds(i * 16, 16)] = zf
        return 0
    lax.fori_loop(0, 40, _fill, 0)
    for i in range(8):
        ones[pl.ds(i * 16, 16)] = of
    for i in range(_ZR // 16):
        pltpu.sync_copy(xsl.at[pl.ds(_N * _S, 16), :],
                        zrows.at[pl.ds(i * 16, 16), :])

    if compute_w:
        def _zw(m, _):
            pltpu.sync_copy(zbuf, w_sh.at[pl.ds(t * _EZT + m * 640, 640)])
            return 0
        lax.fori_loop(0, _EZT // 640, _zw, 0)
        plsc.subcore_barrier()

        def _cnt(k, _):
            r0 = t * (_PPT // 128) + k * _IDXR
            pltpu.sync_copy(eidx.at[pl.ds(r0, _IDXR), :], ei2)
            hs = [pltpu.async_copy(ones.at[pl.ds(0, 128)],
                                   w_sh.at[ei2.at[j]], sem2, add=True)
                  for j in range(_IDXR)]
            for h in hs:
                h.wait()
            return 0
        lax.fori_loop(0, _NCHUNK, _cnt, 0)
        plsc.subcore_barrier()

        def _wc(k, _):
            b = t * _ECHT + k * _ECH
            pltpu.sync_copy(w_sh.at[pl.ds(b, _ECH)], wcv.at[pl.ds(0, _ECH)])
            pltpu.sync_copy(dege.at[pl.ds(b, _ECH)], dcv)

            def _wv(j, _):
                cv = wcv[pl.ds(j * 16, 16)]
                dv = dcv[pl.ds(j * 16, 16)]
                wcv[pl.ds(j * 16, 16)] = dv / jnp.maximum(cv, 1.0)
                return 0
            lax.fori_loop(0, _ECH // 16, _wv, 0)
            pltpu.sync_copy(wcv.at[pl.ds(0, _ECH)], w_sh.at[pl.ds(b, _ECH)])
            pltpu.sync_copy(wcv.at[pl.ds(0, _ECH)], w_out.at[pl.ds(b, _ECH)])
            return 0
        lax.fori_loop(0, _NECH, _wc, 0)
        pltpu.sync_copy(zbuf.at[pl.ds(0, 240)],
                        w_sh.at[pl.ds(_E + t * 240, 240)])
        plsc.subcore_barrier()
    else:
        def _wl(k, _):
            b = t * _ECHT + k * _ECH
            pltpu.sync_copy(w_in.at[pl.ds(b, _ECH)], wcv.at[pl.ds(0, _ECH)])
            pltpu.sync_copy(wcv.at[pl.ds(0, _ECH)], w_sh.at[pl.ds(b, _ECH)])
            return 0
        lax.fori_loop(0, _NECH, _wl, 0)
        pltpu.sync_copy(zbuf.at[pl.ds(0, 240)],
                        w_sh.at[pl.ds(_E + t * 240, 240)])
        plsc.subcore_barrier()

    def _slice(s_local, _):
        s = c * 16 + s_local

        def _za(m, _):
            pltpu.sync_copy(zrows, acc_e.at[pl.ds(t * _EZT + m * _ZR, _ZR), :])
            return 0
        lax.fori_loop(0, _EZT // _ZR, _za, 0)
        pltpu.sync_copy(zrows, acc_v.at[pl.ds(t * _AVZT, _ZR), :])
        pltpu.sync_copy(zrows, acc_v.at[pl.ds(t * _AVZT + _ZR, _ZR), :])
        plsc.subcore_barrier()

        off = s

        def _pha(k, _):
            r0 = t * (_PPT // 128) + k * _IDXR
            pltpu.sync_copy(vidx32.at[pl.ds(r0, _IDXR), :], vi2)

            def _adj(r, _):
                for l in range(8):
                    va2[r, pl.ds(l * 16, 16)] = vi2[r, pl.ds(l * 16, 16)] + off
                return 0
            lax.fori_loop(0, _IDXR, _adj, 0)

            hs = [pltpu.async_copy(xsl.at[va2.at[j]],
                                   rows.at[pl.ds(j * 128, 128), :], sem1)
                  for j in range(_IDXR)]
            pltpu.sync_copy(eidx.at[pl.ds(r0, _IDXR), :], ei2)
            hs2 = []
            for j in range(_IDXR):
                hs[j].wait()
                hs2.append(pltpu.async_copy(rows.at[pl.ds(j * 128, 128), :],
                                            acc_e.at[ei2.at[j]], sem2,
                                            add=True))
            for h in hs2:
                h.wait()
            return 0
        lax.fori_loop(0, _NCHUNK, _pha, 0)
        plsc.subcore_barrier()

        def _scl(k, _):
            b = t * _EZT + k * _SCH
            pltpu.sync_copy(w_sh.at[pl.ds(b, _SCH)], wcv.at[pl.ds(0, _SCH)])
            pltpu.sync_copy(acc_e.at[pl.ds(b, 512), :],
                            rws.at[:, pl.ds(0, 8)])
            pltpu.sync_copy(acc_e.at[pl.ds(b + 512, 512), :],
                            rws.at[:, pl.ds(8, 8)])

            def _sv(g, _):
                wa = wcv[pl.ds(g * 16, 16)]
                wb = wcv[pl.ds(512 + g * 16, 16)]
                for kk in range(16):
                    r = g * 16 + kk
                    wv = jnp.where(iota < 8, wa[kk], wb[kk])
                    rws[r, :] = rws[r, :] * wv
                return 0
            lax.fori_loop(0, 32, _sv, 0)
            pltpu.sync_copy(rws.at[:, pl.ds(0, 8)],
                            acc_e.at[pl.ds(b, 512), :])
            pltpu.sync_copy(rws.at[:, pl.ds(8, 8)],
                            acc_e.at[pl.ds(b + 512, 512), :])
            return 0
        lax.fori_loop(0, _EZT // _SCH, _scl, 0)
        plsc.subcore_barrier()

        def _phb(k, _):
            r0 = t * (_PPT // 128) + k * _IDXR
            pltpu.sync_copy(eidx.at[pl.ds(r0, _IDXR), :], ei2)
            hs = [pltpu.async_copy(acc_e.at[ei2.at[j]],
                                   rows.at[pl.ds(j * 128, 128), :], sem1)
                  for j in range(_IDXR)]
            pltpu.sync_copy(vidx.at[pl.ds(r0, _IDXR), :], vi2)
            hs2 = []
            for j in range(_IDXR):
                hs[j].wait()
                hs2.append(pltpu.async_copy(rows.at[pl.ds(j * 128, 128), :],
                                            acc_v.at[vi2.at[j]], sem2,
                                            add=True))
            for h in hs2:
                h.wait()
            return 0
        lax.fori_loop(0, _NCHUNK, _phb, 0)
        plsc.subcore_barrier()

        pltpu.sync_copy(acc_v.at[pl.ds(t * _OV0, _OV0), :],
                        out.at[pl.ds(t * _OV0, _OV0), s, :])

        @pl.when(t == _NS - 1)
        def _tail():
            pltpu.sync_copy(acc_v.at[pl.ds(_NS * _OV0, _OV1 - _OV0), :],
                            out.at[pl.ds(_NS * _OV0, _OV1 - _OV0), s, :])
        plsc.subcore_barrier()
        return 0

    lax.fori_loop(0, 16, _slice, 0)


def _make_sc_kernel(compute_w):
    mesh = plsc.VectorSubcoreMesh(core_axis_name="c", subcore_axis_name="s",
                                  num_cores=_NC, num_subcores=_NS)
    out_type = [jax.ShapeDtypeStruct((_N, _S, _W), jnp.float32)]
    if compute_w:
        out_type.append(jax.ShapeDtypeStruct((_E,), jnp.float32))
    scratch = [
        pltpu.VMEM_SHARED((_EPAD, _W), jnp.float32),
        pltpu.VMEM_SHARED((_AVROWS, _W), jnp.float32),
        pltpu.VMEM_SHARED((_EPAD,), jnp.float32),
        pltpu.VMEM((_CH, _W), jnp.float32),
        pltpu.VMEM((_ZR, _W), jnp.float32),
        pltpu.VMEM((640,), jnp.float32),
        pltpu.VMEM((128,), jnp.float32),
        pltpu.VMEM((_IDXR, 128), jnp.int32),
        pltpu.VMEM((_IDXR, 128), jnp.int32),
        pltpu.VMEM((_IDXR, 128), jnp.int32),
        pltpu.VMEM((_ECH,), jnp.float32),
        pltpu.VMEM((_ECH,), jnp.float32),
        pltpu.VMEM((512, 16), jnp.float32),
        pltpu.SemaphoreType.DMA,
        pltpu.SemaphoreType.DMA,
    ]
    return pl.kernel(functools.partial(_sc_body, compute_w),
                     out_type=tuple(out_type), mesh=mesh,
                     scratch_types=scratch,
                     compiler_params=pltpu.CompilerParams(
                         use_tc_tiling_on_sc=False))


_sc_layer1 = _make_sc_kernel(True)
_sc_layer2 = _make_sc_kernel(False)


def _tc_body(x_ref, xv_ref, dv_ref, w_ref, b_ref, o_ref):
    a = x_ref[...] + xv_ref[...] * dv_ref[...]
    y = jnp.dot(a, w_ref[...], preferred_element_type=jnp.float32)
    y = y + b_ref[...]
    nrm = jnp.sqrt(jnp.sum(y * y, axis=1, keepdims=True)) + 1e-12
    o_ref[...] = jnp.maximum(y / nrm, 0.0)


_TC_BS = 1000


def _tc_layer(x, xv, degv2, w, b):
    grid = (_N // _TC_BS,)
    return pl.pallas_call(
        _tc_body,
        grid=grid,
        in_specs=[
            pl.BlockSpec((_TC_BS, _D), lambda i: (i, 0)),
            pl.BlockSpec((_TC_BS, _D), lambda i: (i, 0)),
            pl.BlockSpec((_TC_BS, 1), lambda i: (i, 0)),
            pl.BlockSpec((_D, _D), lambda i: (0, 0)),
            pl.BlockSpec((1, _D), lambda i: (0, 0)),
        ],
        out_specs=pl.BlockSpec((_TC_BS, _D), lambda i: (i, 0)),
        out_shape=jax.ShapeDtypeStruct((_N, _D), jnp.float32),
    )(x, xv, degv2, w, b)


def _flat_layout(x):
    return jnp.pad(x, ((0, 8), (0, 0))).reshape(_XROWS, _W)


def kernel(vertex, edges, degE, degV, user_emb, item_emb, W1, b1, W2, b2):
    x = jnp.concatenate([user_emb, item_emb], axis=0)

    npad = _NNZP - _NNZ
    pad_i = jnp.arange(npad, dtype=jnp.int32)
    vpad = _N + (pad_i % 8)
    vfull = jnp.concatenate([vertex, vpad])
    vp = vfull.reshape(_NNZP // 128, 128)
    vp32 = (vfull * _S).reshape(_NNZP // 128, 128)
    ep = jnp.concatenate([edges, _E + (pad_i % 2048)]).reshape(_NNZP // 128, 128)
    degv2 = degV.reshape(_N, 1)
    b1r = b1.reshape(1, _D)
    b2r = b2.reshape(1, _D)

    xv1, w = _sc_layer1(_flat_layout(x), vp, vp32, ep, degE)
    x1 = _tc_layer(x, xv1.reshape(_N, _D), degv2, W1, b1r)
    (xv2,) = _sc_layer2(_flat_layout(x1), vp, vp32, ep, w)
    x2 = _tc_layer(x1, xv2.reshape(_N, _D), degv2, W2, b2r)
    return (x2[:_NU], x2[_NU:])
